# baseline (speedup 1.0000x reference)
import functools

import jax
import jax.numpy as jnp
from jax import lax
from jax.experimental import pallas as pl
from jax.experimental.pallas import tpu as pltpu
from jax.experimental.pallas import tpu_sc as plsc

_ROWS = 64
_COLS = 1000000
_CHUNK = 1792
_MAIN = 999936
_NCHUNK = _MAIN // _CHUNK
_TAIL = _COLS - _MAIN
_VECS = _CHUNK // 16
_PER_SHARD = 140
_NBUF = 8
_NUM_CORES = 2
_NUM_SUBCORES = 16
_INT_MAX = 2**31 - 1

_mesh = plsc.VectorSubcoreMesh(
    core_axis_name="c", subcore_axis_name="s",
    num_cores=_NUM_CORES, num_subcores=_NUM_SUBCORES,
)


def _scan_chunk(buf, vec_base, carry):
    def body(i, c):
        ib = lax.broadcast_in_dim(vec_base + i, (16,), ())
        out = []
        for r in range(8):
            bv, bn = c[2 * r], c[2 * r + 1]
            v = buf[r, pl.ds(i * 16, 16)]
            m = v > bv
            out.append(jnp.where(m, v, bv))
            out.append(jnp.where(m, ib, bn))
        return tuple(out)

    return plsc.parallel_loop(0, _VECS, step=1, unroll=2, carry=carry)(body)


_KERNEL_KWARGS = dict(
    out_type=(jax.ShapeDtypeStruct((32, 16), jnp.float32),
              jax.ShapeDtypeStruct((32, 16), jnp.int32)),
    mesh=_mesh,
    scratch_types=(
        [pltpu.VMEM((8, _CHUNK), jnp.float32) for _ in range(_NBUF)]
        + [
            pltpu.VMEM((8, _TAIL), jnp.float32),
            pltpu.VMEM((16,), jnp.float32),
            pltpu.VMEM((16,), jnp.int32),
        ]
        + [pltpu.SemaphoreType.DMA for _ in range(_NBUF)]
        + [pltpu.SemaphoreType.DMA]
    ),
    compiler_params=pltpu.CompilerParams(needs_layout_passes=False),
)


def _argmax_body(x_hbm, oval_hbm, oidx_hbm, *refs):
    bufs = refs[:_NBUF]
    tailbuf, val_v, idx_v = refs[_NBUF:_NBUF + 3]
    sems = refs[_NBUF + 3:2 * _NBUF + 3]
    semt = refs[2 * _NBUF + 3]
    c = lax.axis_index("c")
    s = lax.axis_index("s")
    wid = c * 16 + s
    g = c * 4 + s // 4
    sh = s % 4
    row0 = pl.multiple_of(g * 8, 8)
    lane = lax.iota(jnp.int32, 16)

    def chunk_src(k):
        cn = jnp.minimum(sh + 4 * k, _NCHUNK - 1)
        col = pl.multiple_of(cn * _CHUNK, _CHUNK)
        return x_hbm.at[pl.ds(row0, 8), pl.ds(col, _CHUNK)], cn

    pltpu.async_copy(
        x_hbm.at[pl.ds(row0, 8), pl.ds(_MAIN, _TAIL)], tailbuf, semt)
    for b in range(_NBUF):
        src, _ = chunk_src(b)
        pltpu.async_copy(src, bufs[b], sems[b])

    neg_inf = jnp.full((16,), -jnp.inf, jnp.float32)
    zero = jnp.zeros((16,), jnp.int32)
    carry = (neg_inf, zero) * 8

    def ring(p, carry):
        for b in range(_NBUF):
            k = _NBUF * p + b
            src, cn = chunk_src(k)
            pltpu.make_async_copy(src, bufs[b], sems[b]).wait()
            carry = _scan_chunk(bufs[b], cn * _VECS, carry)

            @pl.when(k + _NBUF < _PER_SHARD)
            def _(k=k, b=b):
                src, _ = chunk_src(k + _NBUF)
                pltpu.async_copy(src, bufs[b], sems[b])

        return carry

    carry = lax.fori_loop(0, _PER_SHARD // _NBUF, ring, carry)

    for b in range(_PER_SHARD % _NBUF):
        k = (_PER_SHARD // _NBUF) * _NBUF + b
        src, cn = chunk_src(k)
        pltpu.make_async_copy(src, bufs[b], sems[b]).wait()
        carry = _scan_chunk(bufs[b], cn * _VECS, carry)

    pltpu.make_async_copy(
        x_hbm.at[pl.ds(row0, 8), pl.ds(_MAIN, _TAIL)], tailbuf, semt).wait()
    carry = list(carry)
    for r in range(8):
        bv, bn = carry[2 * r], carry[2 * r + 1]
        for i in range(_TAIL // 16):
            v = tailbuf[r, pl.ds(i * 16, 16)]
            ib = jnp.full((16,), _MAIN // 16 + i, jnp.int32)
            m = v > bv
            bv = jnp.where(m, v, bv)
            bn = jnp.where(m, ib, bn)
        carry[2 * r], carry[2 * r + 1] = bv, bn

    res_val = jnp.full((16,), -jnp.inf, jnp.float32)
    res_idx = jnp.zeros((16,), jnp.int32)
    for r in range(8):
        bv, bn = carry[2 * r], carry[2 * r + 1]
        idx = (bn << 4) + lane
        mx = jnp.max(bv)
        cand = jnp.where(bv == mx, idx, jnp.int32(_INT_MAX))
        ii = jnp.min(cand)
        res_val = jnp.where(lane == r, mx, res_val)
        res_idx = jnp.where(lane == r, ii, res_idx)

    val_v[...] = res_val
    idx_v[...] = res_idx
    pltpu.sync_copy(val_v, oval_hbm.at[wid])
    pltpu.sync_copy(idx_v, oidx_hbm.at[wid])


_argmax_sc = pl.kernel(_argmax_body, **_KERNEL_KWARGS)


def _merge_body(val_ref, idx_ref, out_ref):
    for g in range(8):
        base = (g // 4) * 16 + (g % 4) * 4
        bv = val_ref[base]
        bi = idx_ref[base]
        for k in range(1, 4):
            ov = val_ref[base + k]
            oi = idx_ref[base + k]
            take = (ov > bv) | ((ov == bv) & (oi < bi))
            bv = jnp.where(take, ov, bv)
            bi = jnp.where(take, oi, bi)
        out_ref[g] = bi


_merge_tc = pl.pallas_call(
    _merge_body,
    out_shape=jax.ShapeDtypeStruct((8, 16), jnp.int32),
)


def kernel(inputs):
    pval, pidx = _argmax_sc(inputs)
    merged = _merge_tc(pval, pidx)
    return merged[:, :8].reshape(_ROWS).astype(jnp.int64)
